# baseline (speedup 1.0000x reference)
import functools

import jax
import jax.numpy as jnp
from jax import lax
from jax.experimental import pallas as pl
from jax.experimental.pallas import tpu as pltpu
from jax.experimental.pallas import tpu_sc as plsc

NUM_LATENTS = 1000000
Z_DIM = 64
BATCH = 16384

NC, NS = 2, 16
NW = NC * NS
B_PER_W = BATCH // NW
CHUNK = 128
NCHUNK = B_PER_W // CHUNK
WINDOW = 16


def _gather_kernel(zt_hbm, idx_hbm, out_hbm, idx_v, rows_v, sem):
    wid = lax.axis_index("s") * NC + lax.axis_index("c")
    base = wid * B_PER_W
    pltpu.sync_copy(idx_hbm.at[pl.ds(base, B_PER_W)], idx_v)

    def body(d, carry):
        for j in range(NCHUNK):
            pltpu.async_copy(
                zt_hbm.at[d].at[idx_v.at[pl.ds(j * CHUNK, CHUNK)]],
                rows_v.at[d, pl.ds(j * CHUNK, CHUNK)],
                sem,
            )

        @pl.when(d >= WINDOW)
        def _drain_one():
            pltpu.make_async_copy(
                zt_hbm.at[0, pl.ds(0, B_PER_W)], rows_v.at[d - WINDOW], sem
            ).wait()

        return carry

    lax.fori_loop(0, Z_DIM, body, 0)
    pltpu.make_async_copy(
        zt_hbm.at[pl.ds(0, WINDOW), pl.ds(0, B_PER_W)],
        rows_v.at[pl.ds(Z_DIM - WINDOW, WINDOW)],
        sem,
    ).wait()
    pltpu.sync_copy(rows_v, out_hbm.at[:, pl.ds(base, B_PER_W)])


@jax.jit
def kernel(Z, indices):
    idx = indices.astype(jnp.int32)
    mesh = plsc.VectorSubcoreMesh(
        core_axis_name="c", subcore_axis_name="s",
        num_cores=NC, num_subcores=NS,
    )
    run = pl.kernel(
        _gather_kernel,
        out_type=jax.ShapeDtypeStruct((Z_DIM, BATCH), jnp.float32),
        mesh=mesh,
        scratch_types=[
            pltpu.VMEM((B_PER_W,), jnp.int32),
            pltpu.VMEM((Z_DIM, B_PER_W), jnp.float32),
            pltpu.SemaphoreType.DMA,
        ],
        compiler_params=pltpu.CompilerParams(use_tc_tiling_on_sc=False),
    )
    return run(Z.T, idx).T


# speedup vs baseline: 9.0153x; 9.0153x over previous
import functools

import jax
import jax.numpy as jnp
from jax import lax
from jax.experimental import pallas as pl
from jax.experimental.pallas import tpu as pltpu
from jax.experimental.pallas import tpu_sc as plsc

NUM_LATENTS = 1000000
Z_DIM = 64
PAD_DIM = 128
BATCH = 16384

NC, NS = 2, 16
NW = NC * NS
B_PER_W = BATCH // NW
CHUNK = 128
NCHUNK = B_PER_W // CHUNK


def _gather_kernel(zw_hbm, idx_hbm, out_hbm, idx_v, rows_v, sem):
    wid = lax.axis_index("s") * NC + lax.axis_index("c")
    base = wid * B_PER_W
    pltpu.sync_copy(idx_hbm.at[pl.ds(base, B_PER_W)], idx_v)
    for j in range(NCHUNK):
        pltpu.async_copy(
            zw_hbm.at[idx_v.at[pl.ds(j * CHUNK, CHUNK)]],
            rows_v.at[pl.ds(j * CHUNK, CHUNK), :],
            sem,
        )
    pltpu.make_async_copy(zw_hbm.at[pl.ds(0, B_PER_W)], rows_v, sem).wait()
    pltpu.sync_copy(rows_v, out_hbm.at[pl.ds(base, B_PER_W), :])


@jax.jit
def kernel(Z, indices):
    idx = indices.astype(jnp.int32)
    Zwide = jnp.pad(Z, ((0, 0), (0, PAD_DIM - Z_DIM)))
    mesh = plsc.VectorSubcoreMesh(
        core_axis_name="c", subcore_axis_name="s",
        num_cores=NC, num_subcores=NS,
    )
    run = pl.kernel(
        _gather_kernel,
        out_type=jax.ShapeDtypeStruct((BATCH, PAD_DIM), jnp.float32),
        mesh=mesh,
        scratch_types=[
            pltpu.VMEM((B_PER_W,), jnp.int32),
            pltpu.VMEM((B_PER_W, PAD_DIM), jnp.float32),
            pltpu.SemaphoreType.DMA,
        ],
    )
    return run(Zwide, idx)[:, :Z_DIM]
